# baseline (speedup 1.0000x reference)
import functools

import jax
import jax.numpy as jnp
from jax import lax
from jax.experimental import pallas as pl
from jax.experimental.pallas import tpu as pltpu
from jax.experimental.pallas import tpu_sc as plsc

N = 10000
NP = 10240
D = 128
E = 320000

NC = 2
NS = 16
NW = NC * NS
K = 128
CH = 80
EPT = CH * K
EPAD = NW * EPT
RPT = NP // NS

_mesh = plsc.VectorSubcoreMesh(core_axis_name="c", subcore_axis_name="s")


@functools.partial(
    pl.kernel,
    out_type=jax.ShapeDtypeStruct((NC, NP), jnp.float32),
    mesh=_mesh,
    scratch_types=[
        pltpu.VMEM_SHARED((NP,), jnp.float32),
        pltpu.VMEM((CH, K), jnp.int32),
        pltpu.VMEM((CH, K), jnp.float32),
        pltpu.VMEM((RPT,), jnp.float32),
    ],
)
def _deg_kernel(dst_hbm, ew_hbm, out_hbm, dacc, dst_v, ew_v, zv):
    c = lax.axis_index("c")
    s = lax.axis_index("s")
    wid = c * NS + s

    def zfill(i, _):
        zv[pl.ds(i * 16, 16)] = jnp.zeros((16,), jnp.float32)
        return 0
    lax.fori_loop(0, RPT // 16, zfill, 0)
    pltpu.sync_copy(zv, dacc.at[pl.ds(s * RPT, RPT)])
    plsc.subcore_barrier()

    pltpu.sync_copy(dst_hbm.at[wid], dst_v)
    pltpu.sync_copy(ew_hbm.at[wid], ew_v)

    def chunk(i, _):
        pltpu.sync_copy(ew_v.at[i], dacc.at[dst_v.at[i]], add=True)
        return 0
    lax.fori_loop(0, CH, chunk, 0)
    plsc.subcore_barrier()

    pltpu.sync_copy(dacc.at[pl.ds(s * RPT, RPT)],
                    out_hbm.at[c, pl.ds(s * RPT, RPT)])


@functools.partial(
    pl.kernel,
    out_type=jax.ShapeDtypeStruct((NC, NP, D), jnp.float32),
    mesh=_mesh,
    scratch_types=(
        [pltpu.VMEM_SHARED((NP, D), jnp.float32)]
        + [pltpu.VMEM((K,), jnp.int32) for _ in range(4)]
        + [pltpu.VMEM((K,), jnp.int32) for _ in range(4)]
        + [pltpu.VMEM((K,), jnp.float32) for _ in range(4)]
        + [pltpu.VMEM((K, D), jnp.float32) for _ in range(2)]
        + [pltpu.SemaphoreType.DMA for _ in range(8)]
    ),
    name="gcn_edge_agg",
)
def _agg_kernel(g_hbm, src_hbm, dst_hbm, ew_hbm, out_hbm, acc,
                src0, src1, src2, src3, dst0, dst1, dst2, dst3,
                ew0, ew1, ew2, ew3, rowsA, rowsB,
                g0, g1, s0, s1, st0, st1, st2, st3):
    c = lax.axis_index("c")
    s = lax.axis_index("s")
    wid = c * NS + s
    srcs = [src0, src1, src2, src3]
    dsts = [dst0, dst1, dst2, dst3]
    ews = [ew0, ew1, ew2, ew3]
    rows = [rowsA, rowsB]
    gsems = [g0, g1]
    ssems = [s0, s1]
    sts = [st0, st1, st2, st3]

    def zfill(i, _):
        for j in range(D // 16):
            rowsA[i, pl.ds(j * 16, 16)] = jnp.zeros((16,), jnp.float32)
        return 0
    lax.fori_loop(0, K, zfill, 0)
    for r in range(RPT // K):
        pltpu.sync_copy(rowsA, acc.at[pl.ds(s * RPT + r * K, K)])
    plsc.subcore_barrier()

    def stage(i, m):
        pltpu.async_copy(src_hbm.at[wid, i], srcs[m], sts[m])
        pltpu.async_copy(dst_hbm.at[wid, i], dsts[m], sts[m])
        pltpu.async_copy(ew_hbm.at[wid, i], ews[m], sts[m])

    def stage_wait(m):
        pltpu.make_async_copy(src_hbm.at[wid, 0], srcs[m], sts[m]).wait()
        pltpu.make_async_copy(dst_hbm.at[wid, 0], dsts[m], sts[m]).wait()
        pltpu.make_async_copy(ew_hbm.at[wid, 0], ews[m], sts[m]).wait()

    def scale(b, m):
        rb, em = rows[b], ews[m]

        def scale_group(gi, _):
            ewv = em[pl.ds(gi * 16, 16)]
            for l in range(16):
                k = gi * 16 + l
                w = ewv[l]
                for j in range(D // 16):
                    sl = pl.ds(j * 16, 16)
                    rb[k, sl] = rb[k, sl] * w
            return 0
        lax.fori_loop(0, K // 16, scale_group, 0)

    pltpu.sync_copy(src_hbm.at[wid, 0], src0)
    pltpu.sync_copy(dst_hbm.at[wid, 0], dst0)
    pltpu.sync_copy(ew_hbm.at[wid, 0], ew0)
    for m in range(1, 4):
        stage(m, m)

    def gather2(msrc, rb, sem):
        pltpu.async_copy(g_hbm.at[srcs[msrc].at[pl.ds(0, K // 2)]],
                         rb.at[pl.ds(0, K // 2)], sem)
        pltpu.async_copy(g_hbm.at[srcs[msrc].at[pl.ds(K // 2, K // 2)]],
                         rb.at[pl.ds(K // 2, K // 2)], sem)

    gather2(0, rowsA, g0)

    NQ = CH // 4

    def quad(q, _):
        for t in range(4):
            b = t % 2
            o = 1 - b
            m = t
            mp1 = (t + 1) % 4
            mm1 = (t + 3) % 4
            pltpu.make_async_copy(g_hbm.at[srcs[m]], rows[b],
                                  gsems[b]).wait()

            def wait_prev_scatter():
                pltpu.make_async_copy(rows[o], acc.at[dsts[mm1]],
                                      ssems[o]).wait()
            if t == 0:
                pl.when(q > 0)(wait_prev_scatter)
            else:
                wait_prev_scatter()

            def do_stage():
                stage(4 * q + t + 3, mm1)
            if t == 0:
                pl.when(q > 0)(do_stage)
            else:
                pl.when(q < NQ - 1)(do_stage)

            def do_gather():
                stage_wait(mp1)
                gather2(mp1, rows[o], gsems[o])
            if t < 3:
                do_gather()
            else:
                pl.when(q < NQ - 1)(do_gather)
            scale(b, m)
            pltpu.async_copy(rows[b], acc.at[dsts[m]], ssems[b], add=True)
        return 0
    lax.fori_loop(0, NQ, quad, 0)

    pltpu.make_async_copy(rowsB, acc.at[dst3], s1).wait()
    plsc.subcore_barrier()

    for r in range(RPT // K):
        row0 = s * RPT + r * K
        pltpu.sync_copy(acc.at[pl.ds(row0, K)],
                        out_hbm.at[c, pl.ds(row0, K)])


_RB = 1024


def _gemm_body(x_ref, w_ref, out_ref):
    out_ref[...] = jnp.dot(x_ref[...], w_ref[...],
                           preferred_element_type=jnp.float32,
                           precision=lax.Precision.HIGHEST)


def _gemm_tc(x, w):
    return pl.pallas_call(
        _gemm_body,
        out_shape=jax.ShapeDtypeStruct((NP, D), jnp.float32),
        grid=(NP // _RB,),
        in_specs=[
            pl.BlockSpec((_RB, D), lambda i: (i, 0)),
            pl.BlockSpec((D, D), lambda i: (0, 0)),
        ],
        out_specs=pl.BlockSpec((_RB, D), lambda i: (i, 0)),
    )(x, w)


def _disg_body(parts_ref, h_ref, dis_ref, g_ref):
    d = parts_ref[:, 0:1] + parts_ref[:, 1:2] + 1.0
    dis = jnp.broadcast_to(lax.rsqrt(d), (_RB, D))
    dis_ref[...] = dis
    g_ref[...] = dis * h_ref[...]


def _disg_tc(parts_t, h):
    return pl.pallas_call(
        _disg_body,
        out_shape=[jax.ShapeDtypeStruct((NP, D), jnp.float32),
                   jax.ShapeDtypeStruct((NP, D), jnp.float32)],
        grid=(NP // _RB,),
        in_specs=[
            pl.BlockSpec((_RB, NC), lambda i: (i, 0)),
            pl.BlockSpec((_RB, D), lambda i: (i, 0)),
        ],
        out_specs=[pl.BlockSpec((_RB, D), lambda i: (i, 0)),
                   pl.BlockSpec((_RB, D), lambda i: (i, 0))],
    )(parts_t, h)


def _mid_body(p0_ref, p1_ref, g_ref, dis_ref, w_ref, b_ref, out_ref):
    h1 = dis_ref[...] * (p0_ref[...] + p1_ref[...] + g_ref[...]) + b_ref[...]
    h1 = jnp.maximum(h1, 0.0)
    h2 = jnp.dot(h1, w_ref[...],
                 preferred_element_type=jnp.float32,
                 precision=lax.Precision.HIGHEST)
    out_ref[...] = dis_ref[...] * h2


def _mid_tc(p0, p1, g, dis, w2, b1):
    return pl.pallas_call(
        _mid_body,
        out_shape=jax.ShapeDtypeStruct((NP, D), jnp.float32),
        grid=(NP // _RB,),
        in_specs=[
            pl.BlockSpec((_RB, D), lambda i: (i, 0)),
            pl.BlockSpec((_RB, D), lambda i: (i, 0)),
            pl.BlockSpec((_RB, D), lambda i: (i, 0)),
            pl.BlockSpec((_RB, D), lambda i: (i, 0)),
            pl.BlockSpec((D, D), lambda i: (0, 0)),
            pl.BlockSpec((1, D), lambda i: (0, 0)),
        ],
        out_specs=pl.BlockSpec((_RB, D), lambda i: (i, 0)),
    )(p0, p1, g, dis, w2, b1)


def _final_body(p0_ref, p1_ref, g_ref, dis_ref, b_ref, out_ref):
    out_ref[...] = (dis_ref[...] * (p0_ref[...] + p1_ref[...] + g_ref[...])
                    + b_ref[...])


def _final_tc(p0, p1, g, dis, b2):
    return pl.pallas_call(
        _final_body,
        out_shape=jax.ShapeDtypeStruct((NP, D), jnp.float32),
        grid=(NP // _RB,),
        in_specs=[
            pl.BlockSpec((_RB, D), lambda i: (i, 0)),
            pl.BlockSpec((_RB, D), lambda i: (i, 0)),
            pl.BlockSpec((_RB, D), lambda i: (i, 0)),
            pl.BlockSpec((_RB, D), lambda i: (i, 0)),
            pl.BlockSpec((1, D), lambda i: (0, 0)),
        ],
        out_specs=pl.BlockSpec((_RB, D), lambda i: (i, 0)),
    )(p0, p1, g, dis, b2)


@jax.jit
def kernel(x, edge_index, edge_weight, W1, b1, W2, b2):
    pad_idx = N + jnp.arange(EPAD - E, dtype=jnp.int32) % (NP - N)
    src = jnp.concatenate([edge_index[0], pad_idx]).reshape(NW, CH, K)
    dst = jnp.concatenate([edge_index[1], pad_idx]).reshape(NW, CH, K)
    ew = jnp.concatenate(
        [edge_weight, jnp.zeros((EPAD - E,), jnp.float32)]).reshape(NW, CH, K)

    x_pad = jnp.concatenate(
        [x, jnp.zeros((NP - N, D), jnp.float32)], axis=0)

    h1 = _gemm_tc(x_pad, W1)
    deg_parts = _deg_kernel(dst, ew)
    dis, g1 = _disg_tc(deg_parts.T, h1)
    parts1 = _agg_kernel(g1, src, dst, ew)
    g2 = _mid_tc(parts1[0], parts1[1], g1, dis, W2,
                 b1.reshape(1, D))
    parts2 = _agg_kernel(g2, src, dst, ew)
    out = _final_tc(parts2[0], parts2[1], g2, dis, b2.reshape(1, D))
    return out[:N]


# speedup vs baseline: 1.0042x; 1.0042x over previous
import functools

import jax
import jax.numpy as jnp
from jax import lax
from jax.experimental import pallas as pl
from jax.experimental.pallas import tpu as pltpu
from jax.experimental.pallas import tpu_sc as plsc

N = 10000
NP = 10240
D = 128
E = 320000

NC = 2
NS = 16
NW = NC * NS
K = 128
CH = 80
EPT = CH * K
EPAD = NW * EPT
RPT = NP // NS

_mesh = plsc.VectorSubcoreMesh(core_axis_name="c", subcore_axis_name="s")


@functools.partial(
    pl.kernel,
    out_type=jax.ShapeDtypeStruct((NC, NP), jnp.float32),
    mesh=_mesh,
    scratch_types=[
        pltpu.VMEM_SHARED((NP,), jnp.float32),
        pltpu.VMEM((CH, K), jnp.int32),
        pltpu.VMEM((CH, K), jnp.float32),
        pltpu.VMEM((RPT,), jnp.float32),
    ],
)
def _deg_kernel(dst_hbm, ew_hbm, out_hbm, dacc, dst_v, ew_v, zv):
    c = lax.axis_index("c")
    s = lax.axis_index("s")
    wid = c * NS + s

    def zfill(i, _):
        zv[pl.ds(i * 16, 16)] = jnp.zeros((16,), jnp.float32)
        return 0
    lax.fori_loop(0, RPT // 16, zfill, 0)
    pltpu.sync_copy(zv, dacc.at[pl.ds(s * RPT, RPT)])
    plsc.subcore_barrier()

    pltpu.sync_copy(dst_hbm.at[wid], dst_v)
    pltpu.sync_copy(ew_hbm.at[wid], ew_v)

    def chunk(i, _):
        pltpu.sync_copy(ew_v.at[i], dacc.at[dst_v.at[i]], add=True)
        return 0
    lax.fori_loop(0, CH, chunk, 0)
    plsc.subcore_barrier()

    pltpu.sync_copy(dacc.at[pl.ds(s * RPT, RPT)],
                    out_hbm.at[c, pl.ds(s * RPT, RPT)])


@functools.partial(
    pl.kernel,
    out_type=jax.ShapeDtypeStruct((NC, NP, D), jnp.float32),
    mesh=_mesh,
    scratch_types=(
        [pltpu.VMEM_SHARED((NP, D), jnp.float32)]
        + [pltpu.VMEM((K,), jnp.int32) for _ in range(4)]
        + [pltpu.VMEM((K,), jnp.int32) for _ in range(4)]
        + [pltpu.VMEM((K,), jnp.float32) for _ in range(4)]
        + [pltpu.VMEM((K, D), jnp.float32) for _ in range(2)]
        + [pltpu.SemaphoreType.DMA for _ in range(8)]
    ),
    name="gcn_edge_agg",
)
def _agg_kernel(g_hbm, src_hbm, dst_hbm, ew_hbm, out_hbm, acc,
                src0, src1, src2, src3, dst0, dst1, dst2, dst3,
                ew0, ew1, ew2, ew3, rowsA, rowsB,
                g0, g1, s0, s1, st0, st1, st2, st3):
    c = lax.axis_index("c")
    s = lax.axis_index("s")
    wid = c * NS + s
    srcs = [src0, src1, src2, src3]
    dsts = [dst0, dst1, dst2, dst3]
    ews = [ew0, ew1, ew2, ew3]
    rows = [rowsA, rowsB]
    gsems = [g0, g1]
    ssems = [s0, s1]
    sts = [st0, st1, st2, st3]

    def zfill(i, _):
        for j in range(D // 16):
            rowsA[i, pl.ds(j * 16, 16)] = jnp.zeros((16,), jnp.float32)
        return 0
    lax.fori_loop(0, K, zfill, 0)
    for r in range(RPT // K):
        pltpu.sync_copy(rowsA, acc.at[pl.ds(s * RPT + r * K, K)])
    plsc.subcore_barrier()

    def stage(i, m):
        pltpu.async_copy(src_hbm.at[wid, i], srcs[m], sts[m])
        pltpu.async_copy(dst_hbm.at[wid, i], dsts[m], sts[m])
        pltpu.async_copy(ew_hbm.at[wid, i], ews[m], sts[m])

    def stage_wait(m):
        pltpu.make_async_copy(src_hbm.at[wid, 0], srcs[m], sts[m]).wait()
        pltpu.make_async_copy(dst_hbm.at[wid, 0], dsts[m], sts[m]).wait()
        pltpu.make_async_copy(ew_hbm.at[wid, 0], ews[m], sts[m]).wait()

    def scale(b, m):
        rb, em = rows[b], ews[m]

        def scale_group(gi, _):
            ewv = em[pl.ds(gi * 16, 16)]
            for l in range(16):
                k = gi * 16 + l
                w = ewv[l]
                for j in range(D // 16):
                    sl = pl.ds(j * 16, 16)
                    rb[k, sl] = rb[k, sl] * w
            return 0
        lax.fori_loop(0, K // 16, scale_group, 0)

    pltpu.sync_copy(src_hbm.at[wid, 0], src0)
    pltpu.sync_copy(dst_hbm.at[wid, 0], dst0)
    pltpu.sync_copy(ew_hbm.at[wid, 0], ew0)
    for m in range(1, 4):
        stage(m, m)
    pltpu.async_copy(g_hbm.at[src0], rowsA, g0)

    NQ = CH // 4

    def quad(q, _):
        for t in range(4):
            b = t % 2
            o = 1 - b
            m = t
            mp1 = (t + 1) % 4
            mm1 = (t + 3) % 4
            pltpu.make_async_copy(g_hbm.at[srcs[m]], rows[b],
                                  gsems[b]).wait()

            def wait_prev_scatter():
                pltpu.make_async_copy(rows[o], acc.at[dsts[mm1]],
                                      ssems[o]).wait()
            if t == 0:
                pl.when(q > 0)(wait_prev_scatter)
            else:
                wait_prev_scatter()

            def do_stage():
                stage(4 * q + t + 3, mm1)
            if t == 0:
                pl.when(q > 0)(do_stage)
            else:
                pl.when(q < NQ - 1)(do_stage)

            def do_gather():
                stage_wait(mp1)
                pltpu.async_copy(g_hbm.at[srcs[mp1]], rows[o], gsems[o])
            if t < 3:
                do_gather()
            else:
                pl.when(q < NQ - 1)(do_gather)
            scale(b, m)
            pltpu.async_copy(rows[b], acc.at[dsts[m]], ssems[b], add=True)
        return 0
    lax.fori_loop(0, NQ, quad, 0)

    pltpu.make_async_copy(rowsB, acc.at[dst3], s1).wait()
    plsc.subcore_barrier()

    for r in range(RPT // K):
        row0 = s * RPT + r * K
        pltpu.sync_copy(acc.at[pl.ds(row0, K)],
                        out_hbm.at[c, pl.ds(row0, K)])


_RB = 1024


def _gemm_body(x_ref, w_ref, out_ref):
    out_ref[...] = jnp.dot(x_ref[...], w_ref[...],
                           preferred_element_type=jnp.float32,
                           precision=lax.Precision.HIGHEST)


def _gemm_tc(x, w):
    return pl.pallas_call(
        _gemm_body,
        out_shape=jax.ShapeDtypeStruct((NP, D), jnp.float32),
        grid=(NP // _RB,),
        in_specs=[
            pl.BlockSpec((_RB, D), lambda i: (i, 0)),
            pl.BlockSpec((D, D), lambda i: (0, 0)),
        ],
        out_specs=pl.BlockSpec((_RB, D), lambda i: (i, 0)),
    )(x, w)


def _disg_body(parts_ref, h_ref, dis_ref, g_ref):
    d = parts_ref[:, 0:1] + parts_ref[:, 1:2] + 1.0
    dis = jnp.broadcast_to(lax.rsqrt(d), (_RB, D))
    dis_ref[...] = dis
    g_ref[...] = dis * h_ref[...]


def _disg_tc(parts_t, h):
    return pl.pallas_call(
        _disg_body,
        out_shape=[jax.ShapeDtypeStruct((NP, D), jnp.float32),
                   jax.ShapeDtypeStruct((NP, D), jnp.float32)],
        grid=(NP // _RB,),
        in_specs=[
            pl.BlockSpec((_RB, NC), lambda i: (i, 0)),
            pl.BlockSpec((_RB, D), lambda i: (i, 0)),
        ],
        out_specs=[pl.BlockSpec((_RB, D), lambda i: (i, 0)),
                   pl.BlockSpec((_RB, D), lambda i: (i, 0))],
    )(parts_t, h)


def _mid_body(p0_ref, p1_ref, g_ref, dis_ref, w_ref, b_ref, out_ref):
    h1 = dis_ref[...] * (p0_ref[...] + p1_ref[...] + g_ref[...]) + b_ref[...]
    h1 = jnp.maximum(h1, 0.0)
    h2 = jnp.dot(h1, w_ref[...],
                 preferred_element_type=jnp.float32,
                 precision=lax.Precision.HIGHEST)
    out_ref[...] = dis_ref[...] * h2


def _mid_tc(p0, p1, g, dis, w2, b1):
    return pl.pallas_call(
        _mid_body,
        out_shape=jax.ShapeDtypeStruct((NP, D), jnp.float32),
        grid=(NP // _RB,),
        in_specs=[
            pl.BlockSpec((_RB, D), lambda i: (i, 0)),
            pl.BlockSpec((_RB, D), lambda i: (i, 0)),
            pl.BlockSpec((_RB, D), lambda i: (i, 0)),
            pl.BlockSpec((_RB, D), lambda i: (i, 0)),
            pl.BlockSpec((D, D), lambda i: (0, 0)),
            pl.BlockSpec((1, D), lambda i: (0, 0)),
        ],
        out_specs=pl.BlockSpec((_RB, D), lambda i: (i, 0)),
    )(p0, p1, g, dis, w2, b1)


def _final_body(p0_ref, p1_ref, g_ref, dis_ref, b_ref, out_ref):
    out_ref[...] = (dis_ref[...] * (p0_ref[...] + p1_ref[...] + g_ref[...])
                    + b_ref[...])


def _final_tc(p0, p1, g, dis, b2):
    return pl.pallas_call(
        _final_body,
        out_shape=jax.ShapeDtypeStruct((NP, D), jnp.float32),
        grid=(NP // _RB,),
        in_specs=[
            pl.BlockSpec((_RB, D), lambda i: (i, 0)),
            pl.BlockSpec((_RB, D), lambda i: (i, 0)),
            pl.BlockSpec((_RB, D), lambda i: (i, 0)),
            pl.BlockSpec((_RB, D), lambda i: (i, 0)),
            pl.BlockSpec((1, D), lambda i: (0, 0)),
        ],
        out_specs=pl.BlockSpec((_RB, D), lambda i: (i, 0)),
    )(p0, p1, g, dis, b2)


@jax.jit
def kernel(x, edge_index, edge_weight, W1, b1, W2, b2):
    pad_idx = N + jnp.arange(EPAD - E, dtype=jnp.int32) % (NP - N)
    src = jnp.concatenate([edge_index[0], pad_idx]).reshape(NW, CH, K)
    dst = jnp.concatenate([edge_index[1], pad_idx]).reshape(NW, CH, K)
    ew = jnp.concatenate(
        [edge_weight, jnp.zeros((EPAD - E,), jnp.float32)]).reshape(NW, CH, K)

    x_pad = jnp.concatenate(
        [x, jnp.zeros((NP - N, D), jnp.float32)], axis=0)

    h1 = _gemm_tc(x_pad, W1)
    deg_parts = _deg_kernel(dst, ew)
    dis, g1 = _disg_tc(deg_parts.T, h1)
    parts1 = _agg_kernel(g1, src, dst, ew)
    g2 = _mid_tc(parts1[0], parts1[1], g1, dis, W2,
                 b1.reshape(1, D))
    parts2 = _agg_kernel(g2, src, dst, ew)
    out = _final_tc(parts2[0], parts2[1], g2, dis, b2.reshape(1, D))
    return out[:N]
